# baseline (speedup 1.0000x reference)
import functools

import jax
import jax.numpy as jnp
from jax import lax
from jax.experimental import pallas as pl
from jax.experimental.pallas import tpu as pltpu
from jax.experimental.pallas import tpu_sc as plsc

_NC = 2
_NS = 16
_NW = _NC * _NS
_CH = 128


def _sc_gather(table, idxs):
    k = len(idxs)
    dt = table.dtype
    cb = 4 if k == 1 else 2
    rows_it = cb * _CH
    d = table.shape[1]
    nrows = idxs[0].shape[0]
    assert nrows % rows_it == 0
    n_super = nrows // rows_it
    n_iter = -(-n_super // _NW)

    mesh = plsc.VectorSubcoreMesh(core_axis_name="c", subcore_axis_name="s")
    out_t = tuple(jax.ShapeDtypeStruct((nrows, d), dt) for _ in range(k))
    scratch = ([pltpu.VMEM((rows_it,), jnp.int32)] * k
               + [pltpu.VMEM((rows_it, d), dt)] * k
               + [pltpu.SemaphoreType.DMA])

    @functools.partial(pl.kernel, out_type=out_t, mesh=mesh,
                       scratch_types=scratch)
    def kern(*refs):
        tab_h = refs[0]
        idx_h = refs[1:1 + k]
        out_h = refs[1 + k:1 + 2 * k]
        idx_v = refs[1 + 2 * k:1 + 3 * k]
        row_v = refs[1 + 3 * k:1 + 4 * k]
        sem = refs[1 + 4 * k]
        wid = lax.axis_index("s") * _NC + lax.axis_index("c")

        def step(i, carry):
            c = wid + i * _NW

            @pl.when(c < n_super)
            def _():
                base = c * rows_it
                for j in range(k):
                    pltpu.sync_copy(idx_h[j].at[pl.ds(base, rows_it)],
                                    idx_v[j])
                cps = []
                for j in range(k):
                    for q in range(cb):
                        cps.append(pltpu.async_copy(
                            tab_h.at[idx_v[j].at[pl.ds(q * _CH, _CH)]],
                            row_v[j].at[pl.ds(q * _CH, _CH)], sem))
                for cp in cps:
                    cp.wait()
                for j in range(k):
                    pltpu.sync_copy(row_v[j], out_h[j].at[pl.ds(base, rows_it)])

            return carry

        lax.fori_loop(0, n_iter, step, 0)

    res = kern(table, *idxs)
    return res if isinstance(res, tuple) else (res,)


def _tc_mlp(a1, a2, code3, w1s, w1d, w2, t32, b2, block, n_total, g, acc):
    nrows, d = a1.shape
    nblocks = nrows // block
    off = g * nblocks

    def body(*refs):
        (a1_ref, a2_ref, code_ref, w1s_ref, w1d_ref, w2_ref, t_ref,
         b2_ref), o_ref = refs[:8], refs[-1]
        a1b = a1_ref[...]
        a2b = a2_ref[...]
        s = (a1b + a2b).astype(jnp.bfloat16)
        df = jnp.abs(a1b - a2b).astype(jnp.bfloat16)
        code = code_ref[0]
        iot = lax.broadcasted_iota(jnp.int32, (32, block), 0)
        oh = (iot == code).astype(jnp.float32)
        f = lax.dot_general(oh, t_ref[...], (((0,), (0,)), ((), ())),
                            preferred_element_type=jnp.float32)
        h = f
        h = h + jnp.dot(s, w1s_ref[...], preferred_element_type=jnp.float32)
        h = h + jnp.dot(df, w1d_ref[...], preferred_element_type=jnp.float32)
        h = jnp.maximum(h, 0.0)
        o_ref[...] = (jnp.dot(h.astype(jnp.bfloat16), w2_ref[...],
                              preferred_element_type=jnp.float32)
                      + b2_ref[...])

    full = lambda i: (0, 0)
    in_specs = [
        pl.BlockSpec((block, d), lambda i: (i, 0)),
        pl.BlockSpec((block, d), lambda i: (i, 0)),
        pl.BlockSpec((1, 1, block), lambda i: (i, 0, 0)),
        pl.BlockSpec((d, d), full),
        pl.BlockSpec((d, d), full),
        pl.BlockSpec((d, d), full),
        pl.BlockSpec((32, d), full),
        pl.BlockSpec((1, d), full),
    ]
    args = [a1, a2, code3, w1s, w1d, w2, t32, b2]
    aliases = {}
    if acc is not None:
        in_specs.append(pl.BlockSpec(memory_space=pl.ANY))
        args.append(acc)
        aliases = {8: 0}
    return pl.pallas_call(
        body,
        grid=(nblocks,),
        in_specs=in_specs,
        out_specs=pl.BlockSpec((block, d), lambda i: (i + off, 0)),
        out_shape=jax.ShapeDtypeStruct((n_total, d), jnp.float32),
        input_output_aliases=aliases,
    )(*args)


def kernel(product_atom_vecs, cand_bond_types, cand_bond_atom_idxs,
           cand_atom_atom_idxs, W1, b1, W2, b2):
    n_cands = cand_atom_atom_idxs.shape[0]
    hidden = product_atom_vecs.shape[1]
    block = 2560
    groups = 5
    ng = n_cands // groups

    ia = cand_atom_atom_idxs.astype(jnp.int32)
    i1 = cand_bond_atom_idxs[:, 0].astype(jnp.int32)
    i2 = cand_bond_atom_idxs[:, 1].astype(jnp.int32)

    bt = cand_bond_types.astype(jnp.int32)
    code = bt[:, 0] + 4 * bt[:, 1] + 8 * bt[:, 2] + 16 * bt[:, 3]
    cs = jnp.arange(32)
    t32 = (W1[cs % 4] + W1[4 + (cs // 4) % 2] + W1[6 + (cs // 8) % 2]
           + W1[8 + cs // 16] + b1[None, :])

    w1s = W1[10:10 + hidden].astype(jnp.bfloat16)
    w1d = W1[10 + hidden:10 + 2 * hidden].astype(jnp.bfloat16)
    w2b = W2.astype(jnp.bfloat16)
    b2r = b2.reshape(1, hidden)

    acc = None
    for g in range(groups):
        sl = slice(g * ng, (g + 1) * ng)
        a1g, a2g = _sc_gather(product_atom_vecs, (i1[sl], i2[sl]))
        code3 = code[sl].reshape(ng // block, 1, block)
        acc = _tc_mlp(a1g, a2g, code3, w1s, w1d, w2b, t32, b2r, block,
                      n_cands, g, acc)

    (cand_atoms_embeds,) = _sc_gather(product_atom_vecs, (ia,))
    return (cand_atoms_embeds, acc)


# speedup vs baseline: 1.0607x; 1.0607x over previous
import functools

import jax
import jax.numpy as jnp
from jax import lax
from jax.experimental import pallas as pl
from jax.experimental.pallas import tpu as pltpu
from jax.experimental.pallas import tpu_sc as plsc

_NC = 2
_NS = 16
_NW = _NC * _NS
_CH = 128


def _sc_gather(table, idxs):
    k = len(idxs)
    dt = table.dtype
    cb = 2 if k == 1 else 1
    rows_it = cb * _CH
    d = table.shape[1]
    nrows = idxs[0].shape[0]
    assert nrows % rows_it == 0
    n_super = nrows // rows_it
    n_iter = -(-n_super // _NW)

    mesh = plsc.VectorSubcoreMesh(core_axis_name="c", subcore_axis_name="s")
    out_t = tuple(jax.ShapeDtypeStruct((nrows, d), dt) for _ in range(k))
    scratch = ([pltpu.VMEM((2, rows_it), jnp.int32)] * k
               + [pltpu.VMEM((2, rows_it, d), dt)] * k
               + [pltpu.SemaphoreType.DMA] * 3)

    @functools.partial(pl.kernel, out_type=out_t, mesh=mesh,
                       scratch_types=scratch)
    def kern(*refs):
        tab_h = refs[0]
        idx_h = refs[1:1 + k]
        out_h = refs[1 + k:1 + 2 * k]
        idx_v = refs[1 + 2 * k:1 + 3 * k]
        row_v = refs[1 + 3 * k:1 + 4 * k]
        sem_i, sem_g, sem_w = refs[1 + 4 * k:4 + 4 * k]
        wid = lax.axis_index("s") * _NC + lax.axis_index("c")

        def wait_writeout(j, b):
            pltpu.make_async_copy(row_v[j].at[b],
                                  out_h[j].at[pl.ds(0, rows_it)],
                                  sem_w).wait()

        @pl.when(wid < n_super)
        def _():
            for j in range(k):
                pltpu.async_copy(idx_h[j].at[pl.ds(wid * rows_it, rows_it)],
                                 idx_v[j].at[0], sem_i)

        def step(i, carry):
            b = lax.rem(i, 2)
            c = wid + i * _NW
            c_prev2 = c - 2 * _NW
            c_next = c + _NW

            @pl.when(c_prev2 >= 0)
            def _():
                for j in range(k):
                    wait_writeout(j, b)

            @pl.when(c < n_super)
            def _():
                for j in range(k):
                    pltpu.make_async_copy(idx_h[j].at[pl.ds(0, rows_it)],
                                          idx_v[j].at[0], sem_i).wait()
                for j in range(k):
                    for q in range(cb):
                        pltpu.async_copy(
                            tab_h.at[idx_v[j].at[b, pl.ds(q * _CH, _CH)]],
                            row_v[j].at[b, pl.ds(q * _CH, _CH)], sem_g)

            @pl.when(c_next < n_super)
            def _():
                for j in range(k):
                    pltpu.async_copy(
                        idx_h[j].at[pl.ds(c_next * rows_it, rows_it)],
                        idx_v[j].at[1 - b], sem_i)

            @pl.when(c < n_super)
            def _():
                for j in range(k):
                    pltpu.make_async_copy(tab_h.at[idx_v[j].at[b]],
                                          row_v[j].at[b], sem_g).wait()
                for j in range(k):
                    pltpu.async_copy(row_v[j].at[b],
                                     out_h[j].at[pl.ds(c * rows_it, rows_it)],
                                     sem_w)

            return carry

        lax.fori_loop(0, n_iter, step, 0)

        for t in (n_iter - 2, n_iter - 1):
            if t >= 0:
                c_t = wid + t * _NW

                @pl.when(c_t < n_super)
                def _():
                    for j in range(k):
                        wait_writeout(j, t % 2)

    res = kern(table, *idxs)
    return res if isinstance(res, tuple) else (res,)


def _tc_mlp(a1, a2, code3, w1s, w1d, w2, t32, b2, block, n_total, g, acc):
    nrows, d = a1.shape
    nblocks = nrows // block
    off = g * nblocks

    def body(*refs):
        (a1_ref, a2_ref, code_ref, w1s_ref, w1d_ref, w2_ref, t_ref,
         b2_ref), o_ref = refs[:8], refs[-1]
        a1b = a1_ref[...]
        a2b = a2_ref[...]
        s = (a1b + a2b).astype(jnp.bfloat16)
        df = jnp.abs(a1b - a2b).astype(jnp.bfloat16)
        code = code_ref[0]
        iot = lax.broadcasted_iota(jnp.int32, (32, block), 0)
        oh = (iot == code).astype(jnp.float32)
        f = lax.dot_general(oh, t_ref[...], (((0,), (0,)), ((), ())),
                            preferred_element_type=jnp.float32)
        h = f
        h = h + jnp.dot(s, w1s_ref[...], preferred_element_type=jnp.float32)
        h = h + jnp.dot(df, w1d_ref[...], preferred_element_type=jnp.float32)
        h = jnp.maximum(h, 0.0)
        o_ref[...] = (jnp.dot(h.astype(jnp.bfloat16), w2_ref[...],
                              preferred_element_type=jnp.float32)
                      + b2_ref[...])

    full = lambda i: (0, 0)
    in_specs = [
        pl.BlockSpec((block, d), lambda i: (i, 0)),
        pl.BlockSpec((block, d), lambda i: (i, 0)),
        pl.BlockSpec((1, 1, block), lambda i: (i, 0, 0)),
        pl.BlockSpec((d, d), full),
        pl.BlockSpec((d, d), full),
        pl.BlockSpec((d, d), full),
        pl.BlockSpec((32, d), full),
        pl.BlockSpec((1, d), full),
    ]
    args = [a1, a2, code3, w1s, w1d, w2, t32, b2]
    aliases = {}
    if acc is not None:
        in_specs.append(pl.BlockSpec(memory_space=pl.ANY))
        args.append(acc)
        aliases = {8: 0}
    return pl.pallas_call(
        body,
        grid=(nblocks,),
        in_specs=in_specs,
        out_specs=pl.BlockSpec((block, d), lambda i: (i + off, 0)),
        out_shape=jax.ShapeDtypeStruct((n_total, d), jnp.float32),
        input_output_aliases=aliases,
    )(*args)


def kernel(product_atom_vecs, cand_bond_types, cand_bond_atom_idxs,
           cand_atom_atom_idxs, W1, b1, W2, b2):
    n_cands = cand_atom_atom_idxs.shape[0]
    hidden = product_atom_vecs.shape[1]
    block = 1280
    groups = 5
    ng = n_cands // groups

    ia = cand_atom_atom_idxs.astype(jnp.int32)
    i1 = cand_bond_atom_idxs[:, 0].astype(jnp.int32)
    i2 = cand_bond_atom_idxs[:, 1].astype(jnp.int32)

    bt = cand_bond_types.astype(jnp.int32)
    code = bt[:, 0] + 4 * bt[:, 1] + 8 * bt[:, 2] + 16 * bt[:, 3]
    cs = jnp.arange(32)
    t32 = (W1[cs % 4] + W1[4 + (cs // 4) % 2] + W1[6 + (cs // 8) % 2]
           + W1[8 + cs // 16] + b1[None, :])

    w1s = W1[10:10 + hidden].astype(jnp.bfloat16)
    w1d = W1[10 + hidden:10 + 2 * hidden].astype(jnp.bfloat16)
    w2b = W2.astype(jnp.bfloat16)
    b2r = b2.reshape(1, hidden)

    acc = None
    for g in range(groups):
        sl = slice(g * ng, (g + 1) * ng)
        a1g, a2g = _sc_gather(product_atom_vecs, (i1[sl], i2[sl]))
        code3 = code[sl].reshape(ng // block, 1, block)
        acc = _tc_mlp(a1g, a2g, code3, w1s, w1d, w2b, t32, b2r, block,
                      n_cands, g, acc)

    (cand_atoms_embeds,) = _sc_gather(product_atom_vecs, (ia,))
    return (cand_atoms_embeds, acc)


# speedup vs baseline: 1.0621x; 1.0014x over previous
import functools

import jax
import jax.numpy as jnp
from jax import lax
from jax.experimental import pallas as pl
from jax.experimental.pallas import tpu as pltpu
from jax.experimental.pallas import tpu_sc as plsc

_NC = 2
_NS = 16
_NW = _NC * _NS
_CH = 128


def _sc_gather(table, idxs):
    k = len(idxs)
    dt = table.dtype
    cb = 2 if k == 1 else 1
    rows_it = cb * _CH
    d = table.shape[1]
    nrows = idxs[0].shape[0]
    assert nrows % rows_it == 0
    n_super = nrows // rows_it
    n_iter = -(-n_super // _NW)

    mesh = plsc.VectorSubcoreMesh(core_axis_name="c", subcore_axis_name="s")
    out_t = tuple(jax.ShapeDtypeStruct((nrows, d), dt) for _ in range(k))
    scratch = ([pltpu.VMEM((2, rows_it), jnp.int32)] * k
               + [pltpu.VMEM((2, rows_it, d), dt)] * k
               + [pltpu.SemaphoreType.DMA] * 3)

    @functools.partial(pl.kernel, out_type=out_t, mesh=mesh,
                       scratch_types=scratch)
    def kern(*refs):
        tab_h = refs[0]
        idx_h = refs[1:1 + k]
        out_h = refs[1 + k:1 + 2 * k]
        idx_v = refs[1 + 2 * k:1 + 3 * k]
        row_v = refs[1 + 3 * k:1 + 4 * k]
        sem_i, sem_g, sem_w = refs[1 + 4 * k:4 + 4 * k]
        wid = lax.axis_index("s") * _NC + lax.axis_index("c")

        def wait_writeout(j, b):
            pltpu.make_async_copy(row_v[j].at[b],
                                  out_h[j].at[pl.ds(0, rows_it)],
                                  sem_w).wait()

        @pl.when(wid < n_super)
        def _():
            for j in range(k):
                pltpu.async_copy(idx_h[j].at[pl.ds(wid * rows_it, rows_it)],
                                 idx_v[j].at[0], sem_i)

        def step(i, carry):
            b = lax.rem(i, 2)
            c = wid + i * _NW
            c_prev2 = c - 2 * _NW
            c_next = c + _NW

            @pl.when(c_prev2 >= 0)
            def _():
                for j in range(k):
                    wait_writeout(j, b)

            @pl.when(c < n_super)
            def _():
                for j in range(k):
                    pltpu.make_async_copy(idx_h[j].at[pl.ds(0, rows_it)],
                                          idx_v[j].at[0], sem_i).wait()
                for j in range(k):
                    for q in range(cb):
                        pltpu.async_copy(
                            tab_h.at[idx_v[j].at[b, pl.ds(q * _CH, _CH)]],
                            row_v[j].at[b, pl.ds(q * _CH, _CH)], sem_g)

            @pl.when(c_next < n_super)
            def _():
                for j in range(k):
                    pltpu.async_copy(
                        idx_h[j].at[pl.ds(c_next * rows_it, rows_it)],
                        idx_v[j].at[1 - b], sem_i)

            @pl.when(c < n_super)
            def _():
                for j in range(k):
                    pltpu.make_async_copy(tab_h.at[idx_v[j].at[b]],
                                          row_v[j].at[b], sem_g).wait()
                for j in range(k):
                    pltpu.async_copy(row_v[j].at[b],
                                     out_h[j].at[pl.ds(c * rows_it, rows_it)],
                                     sem_w)

            return carry

        lax.fori_loop(0, n_iter, step, 0)

        for t in (n_iter - 2, n_iter - 1):
            if t >= 0:
                c_t = wid + t * _NW

                @pl.when(c_t < n_super)
                def _():
                    for j in range(k):
                        wait_writeout(j, t % 2)

    res = kern(table, *idxs)
    return res if isinstance(res, tuple) else (res,)


def _tc_mlp(a1, a2, code3, w1s, w1d, w2, t32, b2, block, n_total, g, acc):
    nrows, d = a1.shape
    nblocks = nrows // block
    off = g * nblocks

    def body(*refs):
        (a1_ref, a2_ref, code_ref, w1s_ref, w1d_ref, w2_ref, t_ref,
         b2_ref), o_ref = refs[:8], refs[-1]
        a1b = a1_ref[...]
        a2b = a2_ref[...]
        s = (a1b + a2b).astype(jnp.bfloat16)
        df = jnp.abs(a1b - a2b).astype(jnp.bfloat16)
        code = code_ref[0]
        iot = lax.broadcasted_iota(jnp.int32, (32, block), 0)
        oh = (iot == code).astype(jnp.float32)
        f = lax.dot_general(oh, t_ref[...], (((0,), (0,)), ((), ())),
                            preferred_element_type=jnp.float32)
        h = f
        h = h + jnp.dot(s, w1s_ref[...], preferred_element_type=jnp.float32)
        h = h + jnp.dot(df, w1d_ref[...], preferred_element_type=jnp.float32)
        h = jnp.maximum(h, 0.0)
        o_ref[...] = (jnp.dot(h.astype(jnp.bfloat16), w2_ref[...],
                              preferred_element_type=jnp.float32)
                      + b2_ref[...])

    full = lambda i: (0, 0)
    in_specs = [
        pl.BlockSpec((block, d), lambda i: (i, 0)),
        pl.BlockSpec((block, d), lambda i: (i, 0)),
        pl.BlockSpec((1, 1, block), lambda i: (i, 0, 0)),
        pl.BlockSpec((d, d), full),
        pl.BlockSpec((d, d), full),
        pl.BlockSpec((d, d), full),
        pl.BlockSpec((32, d), full),
        pl.BlockSpec((1, d), full),
    ]
    args = [a1, a2, code3, w1s, w1d, w2, t32, b2]
    aliases = {}
    if acc is not None:
        in_specs.append(pl.BlockSpec(memory_space=pl.ANY))
        args.append(acc)
        aliases = {8: 0}
    return pl.pallas_call(
        body,
        grid=(nblocks,),
        in_specs=in_specs,
        out_specs=pl.BlockSpec((block, d), lambda i: (i + off, 0)),
        out_shape=jax.ShapeDtypeStruct((n_total, d), jnp.float32),
        input_output_aliases=aliases,
    )(*args)


def kernel(product_atom_vecs, cand_bond_types, cand_bond_atom_idxs,
           cand_atom_atom_idxs, W1, b1, W2, b2):
    n_cands = cand_atom_atom_idxs.shape[0]
    hidden = product_atom_vecs.shape[1]
    block = 1280
    groups = 10
    ng = n_cands // groups

    ia = cand_atom_atom_idxs.astype(jnp.int32)
    i1 = cand_bond_atom_idxs[:, 0].astype(jnp.int32)
    i2 = cand_bond_atom_idxs[:, 1].astype(jnp.int32)

    bt = cand_bond_types.astype(jnp.int32)
    code = bt[:, 0] + 4 * bt[:, 1] + 8 * bt[:, 2] + 16 * bt[:, 3]
    cs = jnp.arange(32)
    t32 = (W1[cs % 4] + W1[4 + (cs // 4) % 2] + W1[6 + (cs // 8) % 2]
           + W1[8 + cs // 16] + b1[None, :])

    w1s = W1[10:10 + hidden].astype(jnp.bfloat16)
    w1d = W1[10 + hidden:10 + 2 * hidden].astype(jnp.bfloat16)
    w2b = W2.astype(jnp.bfloat16)
    b2r = b2.reshape(1, hidden)

    acc = None
    for g in range(groups):
        sl = slice(g * ng, (g + 1) * ng)
        a1g, a2g = _sc_gather(product_atom_vecs, (i1[sl], i2[sl]))
        code3 = code[sl].reshape(ng // block, 1, block)
        acc = _tc_mlp(a1g, a2g, code3, w1s, w1d, w2b, t32, b2r, block,
                      n_cands, g, acc)

    (cand_atoms_embeds,) = _sc_gather(product_atom_vecs, (ia,))
    return (cand_atoms_embeds, acc)
